# baseline (speedup 1.0000x reference)
import jax
import jax.numpy as jnp
from jax.experimental import pallas as pl
from jax.experimental.pallas import tpu as pltpu


def _up_kernel(x_ref, w_ref, b_ref, br_ref, o_ref):
    H, W, Cin = x_ref.shape
    C2 = o_ref.shape[3]
    x = x_ref[...].reshape(H * W, Cin)
    y = jnp.dot(x, w_ref[...], preferred_element_type=jnp.float32) + b_ref[...]
    for di in range(2):
        ydi = y[:, di * C2:(di + 1) * C2].reshape(H, 1, W, C2)
        o_ref[:, di:di + 1, :, :] = ydi + br_ref[:, di:di + 1, :, :]


def _upsample_add(x_nhwc, w_t, b, bridge_nhwc):
    N, H, W, Cin = x_nhwc.shape
    Cout = w_t.shape[1]
    C2 = 2 * Cout
    w4 = jnp.transpose(w_t, (0, 2, 3, 1)).reshape(Cin, 4 * Cout).astype(jnp.bfloat16)
    b4 = jnp.tile(b, 4).reshape(1, 4 * Cout)
    br_v = bridge_nhwc.reshape(N, H, 2, W, C2)

    y = pl.pallas_call(
        _up_kernel,
        out_shape=jax.ShapeDtypeStruct((N, H, 2, W, C2), jnp.float32),
        grid=(N,),
        in_specs=[
            pl.BlockSpec((None, H, W, Cin), lambda n: (n, 0, 0, 0)),
            pl.BlockSpec((Cin, 4 * Cout), lambda n: (0, 0)),
            pl.BlockSpec((1, 4 * Cout), lambda n: (0, 0)),
            pl.BlockSpec((None, H, 2, W, C2), lambda n: (n, 0, 0, 0, 0)),
        ],
        out_specs=pl.BlockSpec((None, H, 2, W, C2), lambda n: (n, 0, 0, 0, 0)),
        compiler_params=pltpu.CompilerParams(dimension_semantics=("parallel",)),
    )(x_nhwc.astype(jnp.bfloat16), w4, b4, br_v)
    return y.reshape(N, 2 * H, 2 * W, Cout)


def _res2_kernel(fm_ref, w1a_ref, b1a_ref, w2a_ref, b2a_ref,
                 w1b_ref, b1b_ref, w2b_ref, b2b_ref, o_ref, apad_ref):
    H2, W2, C = o_ref.shape
    apad_ref[...] = jnp.zeros_like(apad_ref)

    def conv3x3(a, w_ref, b_ref):
        apad_ref[1:H2 + 1, 1:W2 + 1, :] = jnp.where(a >= 0, a, 0.2 * a).astype(jnp.bfloat16)
        acc = jnp.zeros((H2 * W2, C), jnp.float32)
        for ky in range(3):
            for kx in range(3):
                patch = apad_ref[ky:ky + H2, kx:kx + W2, :].reshape(H2 * W2, C)
                acc = acc + jnp.dot(patch, w_ref[ky, kx],
                                    preferred_element_type=jnp.float32)
        return (acc + b_ref[...]).reshape(H2, W2, C)

    fm = fm_ref[...]
    fm = fm + conv3x3(conv3x3(fm, w1a_ref, b1a_ref), w2a_ref, b2a_ref)
    o_ref[...] = fm + conv3x3(conv3x3(fm, w1b_ref, b1b_ref), w2b_ref, b2b_ref)


def _res2(fm_nhwc, w1a, b1a, w2a, b2a, w1b, b1b, w2b, b2b):
    N, H2, W2, C = fm_nhwc.shape
    wspec = pl.BlockSpec((3, 3, C, C), lambda n: (0, 0, 0, 0))
    bspec = pl.BlockSpec((1, C), lambda n: (0, 0))
    bf = jnp.bfloat16
    return pl.pallas_call(
        _res2_kernel,
        out_shape=jax.ShapeDtypeStruct((N, H2, W2, C), jnp.float32),
        grid=(N,),
        in_specs=[
            pl.BlockSpec((None, H2, W2, C), lambda n: (n, 0, 0, 0)),
            wspec, bspec, wspec, bspec, wspec, bspec, wspec, bspec,
        ],
        out_specs=pl.BlockSpec((None, H2, W2, C), lambda n: (n, 0, 0, 0)),
        scratch_shapes=[pltpu.VMEM((H2 + 2, W2 + 2, C), bf)],
        compiler_params=pltpu.CompilerParams(dimension_semantics=("parallel",)),
    )(fm_nhwc, w1a.astype(bf), b1a.reshape(1, C), w2a.astype(bf), b2a.reshape(1, C),
      w1b.astype(bf), b1b.reshape(1, C), w2b.astype(bf), b2b.reshape(1, C))


def kernel(x_nchw, bridge_nchw, up_w, up_b, w1_0, b1_0, w2_0, b2_0,
           w1_1, b1_1, w2_1, b2_1):
    x = jnp.transpose(x_nchw, (0, 2, 3, 1))
    bridge = jnp.transpose(bridge_nchw, (0, 2, 3, 1))
    fm = _upsample_add(x, up_w, up_b, bridge)
    out = _res2(fm, w1_0, b1_0, w2_0, b2_0, w1_1, b1_1, w2_1, b2_1)
    return jnp.transpose(out, (0, 3, 1, 2))


# speedup vs baseline: 1.1708x; 1.1708x over previous
import jax
import jax.numpy as jnp
from jax.experimental import pallas as pl
from jax.experimental.pallas import tpu as pltpu


def _up_kernel(x_ref, w_ref, b_ref, br_ref, o_ref):
    H, W, Cin = x_ref.shape
    C2 = o_ref.shape[3]
    x = x_ref[...].reshape(H * W, Cin)
    y = jnp.dot(x, w_ref[...], preferred_element_type=jnp.float32) + b_ref[...]
    for di in range(2):
        ydi = y[:, di * C2:(di + 1) * C2].reshape(H, 1, W, C2)
        o_ref[:, di:di + 1, :, :] = ydi + br_ref[:, di:di + 1, :, :]


def _upsample_add(x_nhwc, w_t, b, bridge_nhwc):
    N, H, W, Cin = x_nhwc.shape
    Cout = w_t.shape[1]
    C2 = 2 * Cout
    w4 = jnp.transpose(w_t, (0, 2, 3, 1)).reshape(Cin, 4 * Cout).astype(jnp.bfloat16)
    b4 = jnp.tile(b, 4).reshape(1, 4 * Cout)
    br_v = bridge_nhwc.reshape(N, H, 2, W, C2)

    y = pl.pallas_call(
        _up_kernel,
        out_shape=jax.ShapeDtypeStruct((N, H, 2, W, C2), jnp.float32),
        grid=(N,),
        in_specs=[
            pl.BlockSpec((None, H, W, Cin), lambda n: (n, 0, 0, 0)),
            pl.BlockSpec((Cin, 4 * Cout), lambda n: (0, 0)),
            pl.BlockSpec((1, 4 * Cout), lambda n: (0, 0)),
            pl.BlockSpec((None, H, 2, W, C2), lambda n: (n, 0, 0, 0, 0)),
        ],
        out_specs=pl.BlockSpec((None, H, 2, W, C2), lambda n: (n, 0, 0, 0, 0)),
        compiler_params=pltpu.CompilerParams(dimension_semantics=("parallel",)),
    )(x_nhwc.astype(jnp.bfloat16), w4, b4, br_v)
    return y.reshape(N, 2 * H, 2 * W, Cout)


_TAPS = [(ky, kx) for ky in range(3) for kx in range(3)]


def _res2_kernel(fm_ref, w1a_ref, b1a_ref, w2a_ref, b2a_ref,
                 w1b_ref, b1b_ref, w2b_ref, b2b_ref, o_ref, apad_ref):
    H2, W2, C = o_ref.shape
    apad_ref[...] = jnp.zeros_like(apad_ref)

    def write_z(a):
        apad_ref[1:H2 + 1, 1:W2 + 1, :] = jnp.where(a >= 0, a, 0.2 * a).astype(jnp.bfloat16)

    def patch(t):
        ky, kx = _TAPS[t]
        return apad_ref[ky:ky + H2, kx:kx + W2, :].reshape(H2 * W2, C)

    def conv3x3(w_ref, b_ref):
        acc = jnp.zeros((H2 * W2, C), jnp.float32)
        for i in range(4):
            lhs = jnp.concatenate([patch(2 * i), patch(2 * i + 1)], axis=1)
            acc = acc + jnp.dot(lhs, w_ref[2 * i * C:(2 * i + 2) * C, :],
                                preferred_element_type=jnp.float32)
        acc = acc + jnp.dot(patch(8), w_ref[8 * C:, :],
                            preferred_element_type=jnp.float32)
        return (acc + b_ref[...]).reshape(H2, W2, C)

    write_z(fm_ref[...])
    write_z(conv3x3(w1a_ref, b1a_ref))
    o_ref[...] = fm_ref[...] + conv3x3(w2a_ref, b2a_ref)
    write_z(o_ref[...])
    write_z(conv3x3(w1b_ref, b1b_ref))
    o_ref[...] = o_ref[...] + conv3x3(w2b_ref, b2b_ref)


def _res2(fm_nhwc, w1a, b1a, w2a, b2a, w1b, b1b, w2b, b2b):
    N, H2, W2, C = fm_nhwc.shape
    wspec = pl.BlockSpec((9 * C, C), lambda n: (0, 0))
    bspec = pl.BlockSpec((1, C), lambda n: (0, 0))
    bf = jnp.bfloat16

    def wk(w):
        return w.reshape(9 * C, C).astype(bf)

    return pl.pallas_call(
        _res2_kernel,
        out_shape=jax.ShapeDtypeStruct((N, H2, W2, C), jnp.float32),
        grid=(N,),
        in_specs=[
            pl.BlockSpec((None, H2, W2, C), lambda n: (n, 0, 0, 0)),
            wspec, bspec, wspec, bspec, wspec, bspec, wspec, bspec,
        ],
        out_specs=pl.BlockSpec((None, H2, W2, C), lambda n: (n, 0, 0, 0)),
        scratch_shapes=[pltpu.VMEM((H2 + 2, W2 + 2, C), bf)],
        compiler_params=pltpu.CompilerParams(dimension_semantics=("parallel",)),
    )(fm_nhwc, wk(w1a), b1a.reshape(1, C), wk(w2a), b2a.reshape(1, C),
      wk(w1b), b1b.reshape(1, C), wk(w2b), b2b.reshape(1, C))


def kernel(x_nchw, bridge_nchw, up_w, up_b, w1_0, b1_0, w2_0, b2_0,
           w1_1, b1_1, w2_1, b2_1):
    x = jnp.transpose(x_nchw, (0, 2, 3, 1))
    bridge = jnp.transpose(bridge_nchw, (0, 2, 3, 1))
    fm = _upsample_add(x, up_w, up_b, bridge)
    out = _res2(fm, w1_0, b1_0, w2_0, b2_0, w1_1, b1_1, w2_1, b2_1)
    return jnp.transpose(out, (0, 3, 1, 2))


# speedup vs baseline: 1.5670x; 1.3385x over previous
import jax
import jax.numpy as jnp
from jax.experimental import pallas as pl
from jax.experimental.pallas import tpu as pltpu


def _up_kernel(x_ref, w_ref, b_ref, br_ref, o_ref):
    H, W, Cin = x_ref.shape
    C2 = o_ref.shape[3]
    x = x_ref[...].reshape(H * W, Cin)
    y = jnp.dot(x, w_ref[...], preferred_element_type=jnp.float32) + b_ref[...]
    for di in range(2):
        ydi = y[:, di * C2:(di + 1) * C2].reshape(H, 1, W, C2)
        o_ref[:, di:di + 1, :, :] = ydi + br_ref[:, di:di + 1, :, :]


def _upsample_add(x_nhwc, w_t, b, bridge_nhwc):
    N, H, W, Cin = x_nhwc.shape
    Cout = w_t.shape[1]
    C2 = 2 * Cout
    w4 = jnp.transpose(w_t, (0, 2, 3, 1)).reshape(Cin, 4 * Cout).astype(jnp.bfloat16)
    b4 = jnp.tile(b, 4).reshape(1, 4 * Cout)
    br_v = bridge_nhwc.reshape(N, H, 2, W, C2)

    y = pl.pallas_call(
        _up_kernel,
        out_shape=jax.ShapeDtypeStruct((N, H, 2, W, C2), jnp.float32),
        grid=(N,),
        in_specs=[
            pl.BlockSpec((None, H, W, Cin), lambda n: (n, 0, 0, 0)),
            pl.BlockSpec((Cin, 4 * Cout), lambda n: (0, 0)),
            pl.BlockSpec((1, 4 * Cout), lambda n: (0, 0)),
            pl.BlockSpec((None, H, 2, W, C2), lambda n: (n, 0, 0, 0, 0)),
        ],
        out_specs=pl.BlockSpec((None, H, 2, W, C2), lambda n: (n, 0, 0, 0, 0)),
        compiler_params=pltpu.CompilerParams(dimension_semantics=("parallel",)),
    )(x_nhwc.astype(jnp.bfloat16), w4, b4, br_v)
    return y.reshape(N, 2 * H, 2 * W, Cout)


def _res2_kernel(fm_ref, w1a_ref, b1a_ref, w2a_ref, b2a_ref,
                 w1b_ref, b1b_ref, w2b_ref, b2b_ref, o_ref, apad_ref, zcat_ref):
    H2, W2, C = o_ref.shape
    apad_ref[...] = jnp.zeros_like(apad_ref)

    def write_z(a):
        apad_ref[1:H2 + 1, 1:W2 + 1, :] = jnp.where(a >= 0, a, 0.2 * a)
        for kx in range(3):
            zcat_ref[:, :, kx * C:(kx + 1) * C] = (
                apad_ref[:, kx:kx + W2, :].astype(jnp.bfloat16))

    def conv3x3(w_ref, b_ref):
        lhs01 = jnp.concatenate(
            [zcat_ref[0:H2].reshape(H2 * W2, 3 * C),
             zcat_ref[1:H2 + 1].reshape(H2 * W2, 3 * C)], axis=1)
        acc = jnp.dot(lhs01, w_ref[0:6 * C, :],
                      preferred_element_type=jnp.float32)
        acc = acc + jnp.dot(zcat_ref[2:H2 + 2].reshape(H2 * W2, 3 * C),
                            w_ref[6 * C:, :], preferred_element_type=jnp.float32)
        return (acc + b_ref[...]).reshape(H2, W2, C)

    write_z(fm_ref[...])
    write_z(conv3x3(w1a_ref, b1a_ref))
    o_ref[...] = fm_ref[...] + conv3x3(w2a_ref, b2a_ref)
    write_z(o_ref[...])
    write_z(conv3x3(w1b_ref, b1b_ref))
    o_ref[...] = o_ref[...] + conv3x3(w2b_ref, b2b_ref)


def _res2(fm_nhwc, w1a, b1a, w2a, b2a, w1b, b1b, w2b, b2b):
    N, H2, W2, C = fm_nhwc.shape
    wspec = pl.BlockSpec((9 * C, C), lambda n: (0, 0))
    bspec = pl.BlockSpec((1, C), lambda n: (0, 0))
    bf = jnp.bfloat16

    def wk(w):
        return w.reshape(9 * C, C).astype(bf)

    return pl.pallas_call(
        _res2_kernel,
        out_shape=jax.ShapeDtypeStruct((N, H2, W2, C), jnp.float32),
        grid=(N,),
        in_specs=[
            pl.BlockSpec((None, H2, W2, C), lambda n: (n, 0, 0, 0)),
            wspec, bspec, wspec, bspec, wspec, bspec, wspec, bspec,
        ],
        out_specs=pl.BlockSpec((None, H2, W2, C), lambda n: (n, 0, 0, 0)),
        scratch_shapes=[pltpu.VMEM((H2 + 2, W2 + 2, C), jnp.float32),
                        pltpu.VMEM((H2 + 2, W2, 3 * C), bf)],
        compiler_params=pltpu.CompilerParams(dimension_semantics=("parallel",)),
    )(fm_nhwc, wk(w1a), b1a.reshape(1, C), wk(w2a), b2a.reshape(1, C),
      wk(w1b), b1b.reshape(1, C), wk(w2b), b2b.reshape(1, C))


def kernel(x_nchw, bridge_nchw, up_w, up_b, w1_0, b1_0, w2_0, b2_0,
           w1_1, b1_1, w2_1, b2_1):
    x = jnp.transpose(x_nchw, (0, 2, 3, 1))
    bridge = jnp.transpose(bridge_nchw, (0, 2, 3, 1))
    fm = _upsample_add(x, up_w, up_b, bridge)
    out = _res2(fm, w1_0, b1_0, w2_0, b2_0, w1_1, b1_1, w2_1, b2_1)
    return jnp.transpose(out, (0, 3, 1, 2))
